# Initial kernel scaffold; baseline (speedup 1.0000x reference)
import jax
import jax.numpy as jnp
from jax.experimental import pallas as pl


def kernel(x, batch, steps, W1, b1, W2, b2, Wout, bout):
    raise NotImplementedError("write your pallas kernel here")


# speedup vs baseline: 2.2757x; 2.2757x over previous
import jax
import jax.numpy as jnp
from jax.experimental import pallas as pl
from jax.experimental.pallas import tpu as pltpu

_P = 512
_R = 0.05
_TI = 32


def _grad_body(xi_ref, xj_ref, Wa_ref, Wb_ref, b1_ref, w1d_ref, W2_ref,
               b2_ref, wo_ref, C1_ref, C2_ref, gi_ref, gj_ref):
    it = pl.program_id(1)
    xi = xi_ref[0]
    xj = xj_ref[0]
    pos_i = xi[:, 0:2]
    pos_j = xj[:, 0:2]

    A = jnp.dot(xi, Wa_ref[...], preferred_element_type=jnp.float32) + b1_ref[...]
    B = jnp.dot(xj, Wb_ref[...], preferred_element_type=jnp.float32)

    sqi = jnp.sum(pos_i * pos_i, axis=1)
    sqj = jnp.sum(pos_j * pos_j, axis=1)
    dotm = jnp.dot(pos_i, pos_j.T, preferred_element_type=jnp.float32)
    dist2 = sqi[:, None] + sqj[None, :] - 2.0 * dotm
    rows = it * _TI + jax.lax.broadcasted_iota(jnp.int32, (_TI, _P), 0)
    cols = jax.lax.broadcasted_iota(jnp.int32, (_TI, _P), 1)
    mask = (dist2 < _R * _R) & (rows != cols)

    relx = pos_i[:, 0:1] - pos_j[:, 0].reshape(1, _P)
    rely = pos_i[:, 1:2] - pos_j[:, 1].reshape(1, _P)
    dist = jnp.sqrt(relx * relx + rely * rely + 1e-8)

    w1d = w1d_ref[...]
    z1 = (A[:, None, :] + B[None, :, :]
          + dist[:, :, None] * w1d[0][None, None, :])
    h = jnp.tanh(z1).reshape(_TI * _P, 32)
    W2 = W2_ref[...]
    z2 = jnp.dot(h, W2, preferred_element_type=jnp.float32) + b2_ref[...]
    t2 = jnp.tanh(z2)
    dz2 = (1.0 - t2 * t2) * wo_ref[...]
    dh = jnp.dot(dz2, W2.T, preferred_element_type=jnp.float32)
    dz1 = dh * (1.0 - h * h)

    s = jnp.sum(dz1 * w1d, axis=1).reshape(_TI, _P)
    t1x = jnp.sum(dz1 * C1_ref[0][None, :], axis=1).reshape(_TI, _P)
    t1y = jnp.sum(dz1 * C1_ref[1][None, :], axis=1).reshape(_TI, _P)
    t2x = jnp.sum(dz1 * C2_ref[0][None, :], axis=1).reshape(_TI, _P)
    t2y = jnp.sum(dz1 * C2_ref[1][None, :], axis=1).reshape(_TI, _P)

    ux = relx / dist
    uy = rely / dist
    sux = s * ux
    suy = s * uy
    gi_x = jnp.sum(jnp.where(mask, t1x + sux, 0.0), axis=1)
    gi_y = jnp.sum(jnp.where(mask, t1y + suy, 0.0), axis=1)
    gj_x = jnp.sum(jnp.where(mask, t2x - sux, 0.0), axis=0)
    gj_y = jnp.sum(jnp.where(mask, t2y - suy, 0.0), axis=0)

    gi_ref[0, 0] = jnp.stack([gi_x, gi_y], axis=0)

    @pl.when(it == 0)
    def _():
        gj_ref[...] = jnp.zeros_like(gj_ref)

    gj_ref[0] = gj_ref[0] + jnp.stack([gj_x, gj_y], axis=0)


def _grad_step(xr, Wa, Wb, b1r, w1d, W2, b2r, wo, C1, C2):
    nb = xr.shape[0]
    grid = (nb, _P // _TI)

    def wspec(shape):
        return pl.BlockSpec(shape, lambda b, it: (0,) * len(shape))

    gi, gj = pl.pallas_call(
        _grad_body,
        grid=grid,
        in_specs=[
            pl.BlockSpec((1, _TI, 4), lambda b, it: (b, it, 0)),
            pl.BlockSpec((1, _P, 4), lambda b, it: (b, 0, 0)),
            wspec(Wa.shape), wspec(Wb.shape), wspec(b1r.shape),
            wspec(w1d.shape), wspec(W2.shape), wspec(b2r.shape),
            wspec(wo.shape), wspec(C1.shape), wspec(C2.shape),
        ],
        out_specs=[
            pl.BlockSpec((1, 1, 2, _TI), lambda b, it: (b, it, 0, 0)),
            pl.BlockSpec((1, 2, _P), lambda b, it: (b, 0, 0)),
        ],
        out_shape=[
            jax.ShapeDtypeStruct((nb, _P // _TI, 2, _TI), jnp.float32),
            jax.ShapeDtypeStruct((nb, 2, _P), jnp.float32),
        ],
        compiler_params=pltpu.CompilerParams(
            dimension_semantics=("parallel", "arbitrary")),
    )(xr, xr, Wa, Wb, b1r, w1d, W2, b2r, wo, C1, C2)
    gi2 = jnp.transpose(gi, (0, 2, 1, 3)).reshape(nb, 2, _P)
    return gi2 + gj


def kernel(x, batch, steps, W1, b1, W2, b2, Wout, bout):
    N = x.shape[0]
    nb = N // _P

    Wr = W1[8:10]
    pad = jnp.zeros((2, W1.shape[1]), dtype=W1.dtype)
    Wa = W1[0:4] + jnp.concatenate([Wr, pad], axis=0)
    Wb = W1[4:8] - jnp.concatenate([Wr, pad], axis=0)
    w1d = W1[10:11]
    C1 = W1[0:2] + Wr
    C2 = W1[4:6] - Wr
    b1r = b1[None, :]
    b2r = b2[None, :]
    wo = Wout[:, 0][None, :]

    def body(_, xc):
        xr = xc.reshape(nb, _P, 4)
        g = _grad_step(xr, Wa, Wb, b1r, w1d, W2, b2r, wo, C1, C2)
        gt = jnp.transpose(g, (0, 2, 1)).reshape(N, 2)
        newx = xc[:, 0:2] - 0.01 * gt
        return jnp.concatenate([newx, xc[:, 2:]], axis=1)

    return jax.lax.fori_loop(0, steps, body, x)


# speedup vs baseline: 13.4810x; 5.9238x over previous
import jax
import jax.numpy as jnp
from jax.experimental import pallas as pl
from jax.experimental.pallas import tpu as pltpu

_P = 512
_R = 0.05
_TI = 32
_F = 32
_C = 4
_L = _F * _C
_Q = _P // _C


def _grad_body(xi_ref, xj4_ref, Wa4_ref, Wb16_ref, sjx_ref, sjy_ref,
               b1_4_ref, w1d4_ref, W2b_ref, W2bT_ref, b2_4_ref, wo4_ref,
               VBS_ref, V1X_ref, V1Y_ref, V2X_ref, V2Y_ref, RED4_ref,
               gi_ref, gj_ref):
    it = pl.program_id(1)
    xi = xi_ref[0]
    xj4 = xj4_ref[0]

    A4 = jnp.dot(xi, Wa4_ref[...], preferred_element_type=jnp.float32) + b1_4_ref[...]
    B4 = jnp.dot(xj4, Wb16_ref[...], preferred_element_type=jnp.float32)

    pix = jnp.broadcast_to(xi[:, 0:1], (_TI, _L))
    piy = jnp.broadcast_to(xi[:, 1:2], (_TI, _L))
    pjx = jnp.dot(xj4, sjx_ref[...], preferred_element_type=jnp.float32)
    pjy = jnp.dot(xj4, sjy_ref[...], preferred_element_type=jnp.float32)

    relx = pix[:, None, :] - pjx[None, :, :]
    rely = piy[:, None, :] - pjy[None, :, :]
    dist2 = ((pix * pix + piy * piy)[:, None, :]
             + (pjx * pjx + pjy * pjy)[None, :, :]
             - 2.0 * (pix[:, None, :] * pjx[None, :, :]
                      + piy[:, None, :] * pjy[None, :, :]))
    j_id = (4 * jax.lax.broadcasted_iota(jnp.int32, (_Q, _L), 0)
            + jax.lax.broadcasted_iota(jnp.int32, (_Q, _L), 1) // _F)
    i_id = it * _TI + jax.lax.broadcasted_iota(jnp.int32, (_TI, _Q, _L), 0)
    mask = (dist2 < _R * _R) & (i_id != j_id[None, :, :])
    dist = jnp.sqrt(relx * relx + rely * rely + 1e-8)
    ux = relx / dist
    uy = rely / dist

    z1 = A4[:, None, :] + B4[None, :, :] + dist * w1d4_ref[...][0][None, None, :]
    h = jnp.tanh(z1).reshape(_TI * _Q, _L)
    z2 = jnp.dot(h, W2b_ref[...], preferred_element_type=jnp.float32) + b2_4_ref[...]
    t2 = jnp.tanh(z2)
    dz2 = (1.0 - t2 * t2) * wo4_ref[...]
    dh = jnp.dot(dz2, W2bT_ref[...], preferred_element_type=jnp.float32)
    dz1 = dh * (1.0 - h * h)

    def red(v_ref):
        r = jnp.dot(dz1, v_ref[...], preferred_element_type=jnp.float32)
        return r.reshape(_TI, _Q, _L)

    s_bc = red(VBS_ref)
    sux = s_bc * ux
    suy = s_bc * uy
    v1x = jnp.where(mask, red(V1X_ref) + sux, 0.0)
    v1y = jnp.where(mask, red(V1Y_ref) + suy, 0.0)
    v2x = jnp.where(mask, red(V2X_ref) - sux, 0.0)
    v2y = jnp.where(mask, red(V2Y_ref) - suy, 0.0)

    gi_x = jnp.sum(v1x, axis=(1, 2)) * (1.0 / _F)
    gi_y = jnp.sum(v1y, axis=(1, 2)) * (1.0 / _F)
    gj2x = jnp.sum(v2x, axis=0)
    gj2y = jnp.sum(v2y, axis=0)
    RED4 = RED4_ref[...]
    gj4 = jnp.concatenate(
        [jnp.dot(gj2x, RED4, preferred_element_type=jnp.float32),
         jnp.dot(gj2y, RED4, preferred_element_type=jnp.float32)], axis=1)

    gi_ref[0, 0] = jnp.stack([gi_x, gi_y], axis=0)

    @pl.when(it == 0)
    def _():
        gj_ref[...] = jnp.zeros_like(gj_ref)

    gj_ref[0] = gj_ref[0] + gj4


def _grad_step(xr, xr4, consts):
    nb = xr.shape[0]
    grid = (nb, _P // _TI)

    def wspec(a):
        return pl.BlockSpec(a.shape, lambda b, it: (0,) * a.ndim)

    gi, gj = pl.pallas_call(
        _grad_body,
        grid=grid,
        in_specs=[
            pl.BlockSpec((1, _TI, 4), lambda b, it: (b, it, 0)),
            pl.BlockSpec((1, _Q, 16), lambda b, it: (b, 0, 0)),
        ] + [wspec(c) for c in consts],
        out_specs=[
            pl.BlockSpec((1, 1, 2, _TI), lambda b, it: (b, it, 0, 0)),
            pl.BlockSpec((1, _Q, 8), lambda b, it: (b, 0, 0)),
        ],
        out_shape=[
            jax.ShapeDtypeStruct((nb, _P // _TI, 2, _TI), jnp.float32),
            jax.ShapeDtypeStruct((nb, _Q, 8), jnp.float32),
        ],
        compiler_params=pltpu.CompilerParams(
            dimension_semantics=("parallel", "arbitrary")),
    )(xr, xr4, *consts)
    return gi, gj


def kernel(x, batch, steps, W1, b1, W2, b2, Wout, bout):
    N = x.shape[0]
    nb = N // _P
    f32 = jnp.float32

    Wr = W1[8:10]
    pad = jnp.zeros((2, _F), dtype=f32)
    Wa = W1[0:4] + jnp.concatenate([Wr, pad], axis=0)
    Wb = W1[4:8] - jnp.concatenate([Wr, pad], axis=0)
    w1d = W1[10:11]
    c1x = W1[0] + W1[8]
    c1y = W1[1] + W1[9]
    c2x = W1[4] - W1[8]
    c2y = W1[5] - W1[9]

    eye4 = jnp.eye(_C, dtype=f32)
    ones1F = jnp.ones((1, _F), dtype=f32)

    def bcmat(vec):
        return jnp.kron(eye4, vec[:, None] @ ones1F)

    e0 = jnp.zeros((4, 1), dtype=f32).at[0, 0].set(1.0)
    e1 = jnp.zeros((4, 1), dtype=f32).at[1, 0].set(1.0)

    consts = (
        jnp.tile(Wa, (1, _C)),
        jnp.kron(eye4, Wb),
        jnp.kron(eye4, e0 @ ones1F),
        jnp.kron(eye4, e1 @ ones1F),
        jnp.tile(b1[None, :], (1, _C)),
        jnp.tile(w1d, (1, _C)),
        jnp.kron(eye4, W2),
        jnp.kron(eye4, W2.T),
        jnp.tile(b2[None, :], (1, _C)),
        jnp.tile(Wout[:, 0][None, :], (1, _C)),
        bcmat(w1d[0]),
        bcmat(c1x), bcmat(c1y), bcmat(c2x), bcmat(c2y),
        jnp.kron(eye4, jnp.ones((_F, 1), dtype=f32) / _F),
    )

    def body(_, xc):
        xr = xc.reshape(nb, _P, 4)
        xr4 = xc.reshape(nb, _Q, 16)
        gi, gj = _grad_step(xr, xr4, consts)
        gix = jnp.transpose(gi, (0, 2, 1, 3)).reshape(nb, 2, _P)
        gjx = gj[:, :, 0:4].reshape(nb, _P)
        gjy = gj[:, :, 4:8].reshape(nb, _P)
        gx = (gix[:, 0] + gjx).reshape(N)
        gy = (gix[:, 1] + gjy).reshape(N)
        newx = xc[:, 0:2] - 0.01 * jnp.stack([gx, gy], axis=1)
        return jnp.concatenate([newx, xc[:, 2:]], axis=1)

    return jax.lax.fori_loop(0, steps, body, x)


# speedup vs baseline: 16.6840x; 1.2376x over previous
import jax
import jax.numpy as jnp
from jax.experimental import pallas as pl
from jax.experimental.pallas import tpu as pltpu

_P = 512
_R = 0.05
_TI = 32
_F = 32
_C = 4
_L = _F * _C
_Q = _P // _C


def _grad_body(xi_ref, xj4_ref, Wa4_ref, Wb16_ref, sjx_ref, sjy_ref,
               b1_4_ref, w1d4_ref, W2b_ref, W2bT_ref, b2_4_ref, wo4_ref,
               VBS_ref, V1X_ref, V1Y_ref, V2X_ref, V2Y_ref, RED4_ref,
               gi_ref, gj_ref):
    it = pl.program_id(1)
    xi = xi_ref[0]
    xj4 = xj4_ref[0]

    A4 = jnp.dot(xi, Wa4_ref[...], preferred_element_type=jnp.float32) + b1_4_ref[...]
    B4 = jnp.dot(xj4, Wb16_ref[...], preferred_element_type=jnp.float32)

    pix = jnp.broadcast_to(xi[:, 0:1], (_TI, _L))
    piy = jnp.broadcast_to(xi[:, 1:2], (_TI, _L))
    pjx = jnp.dot(xj4, sjx_ref[...], preferred_element_type=jnp.float32)
    pjy = jnp.dot(xj4, sjy_ref[...], preferred_element_type=jnp.float32)

    relx = pix[:, None, :] - pjx[None, :, :]
    rely = piy[:, None, :] - pjy[None, :, :]
    dist2 = ((pix * pix + piy * piy)[:, None, :]
             + (pjx * pjx + pjy * pjy)[None, :, :]
             - 2.0 * (pix[:, None, :] * pjx[None, :, :]
                      + piy[:, None, :] * pjy[None, :, :]))
    j_id = (4 * jax.lax.broadcasted_iota(jnp.int32, (_Q, _L), 0)
            + jax.lax.broadcasted_iota(jnp.int32, (_Q, _L), 1) // _F)
    i_id = it * _TI + jax.lax.broadcasted_iota(jnp.int32, (_TI, _Q, _L), 0)
    mask = (dist2 < _R * _R) & (i_id != j_id[None, :, :])
    r2 = relx * relx + rely * rely + 1e-8
    rdist = jax.lax.rsqrt(r2)
    dist = r2 * rdist

    z1 = A4[:, None, :] + B4[None, :, :] + dist * w1d4_ref[...][0][None, None, :]
    h = jnp.tanh(z1).reshape(_TI * _Q, _L)
    z2 = jnp.dot(h, W2b_ref[...], preferred_element_type=jnp.float32) + b2_4_ref[...]
    t2 = jnp.tanh(z2)
    maskf = mask.reshape(_TI * _Q, _L)
    dz2 = jnp.where(maskf, (1.0 - t2 * t2) * wo4_ref[...], 0.0)
    dh = jnp.dot(dz2, W2bT_ref[...], preferred_element_type=jnp.float32)
    dz1 = dh * (1.0 - h * h)

    def red(v_ref):
        r = jnp.dot(dz1, v_ref[...], preferred_element_type=jnp.float32)
        return r.reshape(_TI, _Q, _L)

    srd = red(VBS_ref) * rdist
    sux = srd * relx
    suy = srd * rely
    v1x = red(V1X_ref) + sux
    v1y = red(V1Y_ref) + suy
    v2x = red(V2X_ref) - sux
    v2y = red(V2Y_ref) - suy

    gi_x = jnp.sum(v1x, axis=(1, 2)) * (1.0 / _F)
    gi_y = jnp.sum(v1y, axis=(1, 2)) * (1.0 / _F)
    gj2x = jnp.sum(v2x, axis=0)
    gj2y = jnp.sum(v2y, axis=0)
    RED4 = RED4_ref[...]
    gj4 = jnp.concatenate(
        [jnp.dot(gj2x, RED4, preferred_element_type=jnp.float32),
         jnp.dot(gj2y, RED4, preferred_element_type=jnp.float32)], axis=1)

    gi_ref[0, 0] = jnp.stack([gi_x, gi_y], axis=0)

    @pl.when(it == 0)
    def _():
        gj_ref[...] = jnp.zeros_like(gj_ref)

    gj_ref[0] = gj_ref[0] + gj4


def _grad_step(xr, xr4, consts):
    nb = xr.shape[0]
    grid = (nb, _P // _TI)

    def wspec(a):
        return pl.BlockSpec(a.shape, lambda b, it: (0,) * a.ndim)

    gi, gj = pl.pallas_call(
        _grad_body,
        grid=grid,
        in_specs=[
            pl.BlockSpec((1, _TI, 4), lambda b, it: (b, it, 0)),
            pl.BlockSpec((1, _Q, 16), lambda b, it: (b, 0, 0)),
        ] + [wspec(c) for c in consts],
        out_specs=[
            pl.BlockSpec((1, 1, 2, _TI), lambda b, it: (b, it, 0, 0)),
            pl.BlockSpec((1, _Q, 8), lambda b, it: (b, 0, 0)),
        ],
        out_shape=[
            jax.ShapeDtypeStruct((nb, _P // _TI, 2, _TI), jnp.float32),
            jax.ShapeDtypeStruct((nb, _Q, 8), jnp.float32),
        ],
        compiler_params=pltpu.CompilerParams(
            dimension_semantics=("parallel", "arbitrary")),
    )(xr, xr4, *consts)
    return gi, gj


def kernel(x, batch, steps, W1, b1, W2, b2, Wout, bout):
    N = x.shape[0]
    nb = N // _P
    f32 = jnp.float32

    Wr = W1[8:10]
    pad = jnp.zeros((2, _F), dtype=f32)
    Wa = W1[0:4] + jnp.concatenate([Wr, pad], axis=0)
    Wb = W1[4:8] - jnp.concatenate([Wr, pad], axis=0)
    w1d = W1[10:11]
    c1x = W1[0] + W1[8]
    c1y = W1[1] + W1[9]
    c2x = W1[4] - W1[8]
    c2y = W1[5] - W1[9]

    eye4 = jnp.eye(_C, dtype=f32)
    ones1F = jnp.ones((1, _F), dtype=f32)

    def bcmat(vec):
        return jnp.kron(eye4, vec[:, None] @ ones1F)

    e0 = jnp.zeros((4, 1), dtype=f32).at[0, 0].set(1.0)
    e1 = jnp.zeros((4, 1), dtype=f32).at[1, 0].set(1.0)

    consts = (
        jnp.tile(Wa, (1, _C)),
        jnp.kron(eye4, Wb),
        jnp.kron(eye4, e0 @ ones1F),
        jnp.kron(eye4, e1 @ ones1F),
        jnp.tile(b1[None, :], (1, _C)),
        jnp.tile(w1d, (1, _C)),
        jnp.kron(eye4, W2),
        jnp.kron(eye4, W2.T),
        jnp.tile(b2[None, :], (1, _C)),
        jnp.tile(Wout[:, 0][None, :], (1, _C)),
        bcmat(w1d[0]),
        bcmat(c1x), bcmat(c1y), bcmat(c2x), bcmat(c2y),
        jnp.kron(eye4, jnp.ones((_F, 1), dtype=f32) / _F),
    )

    def body(_, xc):
        xr = xc.reshape(nb, _P, 4)
        xr4 = xc.reshape(nb, _Q, 16)
        gi, gj = _grad_step(xr, xr4, consts)
        gix = jnp.transpose(gi, (0, 2, 1, 3)).reshape(nb, 2, _P)
        gjx = gj[:, :, 0:4].reshape(nb, _P)
        gjy = gj[:, :, 4:8].reshape(nb, _P)
        gx = (gix[:, 0] + gjx).reshape(N)
        gy = (gix[:, 1] + gjy).reshape(N)
        newx = xc[:, 0:2] - 0.01 * jnp.stack([gx, gy], axis=1)
        return jnp.concatenate([newx, xc[:, 2:]], axis=1)

    return jax.lax.fori_loop(0, steps, body, x)
